# baseline (speedup 1.0000x reference)
import functools

import jax
import jax.numpy as jnp
from jax import lax
from jax.experimental import pallas as pl
from jax.experimental.pallas import tpu as pltpu
from jax.experimental.pallas import tpu_sc as plsc

B, W, C, S = 32, 2048, 128, 32
TR = W // 2
U = W - TR
_SQRT2 = 1.4142135623730951
_RC = 256
_NB = W // _RC
_GCHUNK = 128
_NGC = U // _GCHUNK


def _gelu(v):
    return v * 0.5 * (1.0 + lax.erf(v / _SQRT2))


def _softmax(v):
    m = jnp.max(v, axis=-1, keepdims=True)
    e = jnp.exp(v - m)
    return e / jnp.sum(e, axis=-1, keepdims=True)


def _win32(a):
    for k in (1, 2, 4, 8, 16):
        z = jnp.zeros((k, a.shape[1]), a.dtype)
        a = a + jnp.concatenate([z, a[: a.shape[0] - k]], axis=0)
    return a


def _win32_scratch(a, scr):
    scr[:S] = jnp.zeros((S, a.shape[1]), a.dtype)
    scr[S:] = a
    for k in (1, 2, 4, 8, 16):
        scr[S:] = scr[S:] + scr[S - k:S - k + W]
    return scr[S:]


def _bf16_rne(v):
    u = lax.bitcast_convert_type(v, jnp.int32)
    r = u + 0x7FFF + jnp.bitwise_and(lax.shift_right_logical(u, 16), 1)
    return lax.bitcast_convert_type(jnp.bitwise_and(r, jnp.int32(-65536)),
                                    jnp.float32)


def _div_den(a):
    wpos = lax.broadcasted_iota(jnp.int32, (S, 1), 0).astype(jnp.float32)
    den = jnp.minimum(wpos + 1.0, float(S))
    return jnp.concatenate([a[:S] / den, a[S:] * (1.0 / S)], axis=0)


def _select_x_kernel(x_ref, We_ref, be_ref, rank_ref, out_ref, cg_ref, scr_ref):
    xb = x_ref[0]

    out_ref[0] = _gelu(lax.dot_general(
        xb, We_ref[...], (((1,), (0,)), ((), ())),
        preferred_element_type=jnp.float32) + be_ref[...])
    gx = lax.dot_general(xb, xb, (((0,), (0,)), ((), ())),
                         preferred_element_type=jnp.float32)
    cg_ref[0] = _softmax(gx / float(W))

    xq = _bf16_rne(xb)
    x2q = _bf16_rne(xb * xb)
    ltrm = _div_den(_win32_scratch(xq, scr_ref))
    ltr2 = _div_den(_win32_scratch(x2q, scr_ref))
    ltrd = ltr2 - ltrm * ltrm
    num = jnp.sum(ltrd, axis=1, keepdims=True)
    dnm = jnp.sum(ltrm, axis=1, keepdims=True)
    score_c = num / dnm
    score_r = jnp.transpose(score_c)

    ii = lax.broadcasted_iota(jnp.int32, (_RC, _RC), 0)
    jj = lax.broadcasted_iota(jnp.int32, (_RC, _RC), 1)
    tri = (jj < ii).astype(jnp.float32)
    rank_cols = []
    for cb in range(_NB):
        s_i = score_c[cb * _RC:(cb + 1) * _RC]
        acc = jnp.zeros((_RC, _RC), jnp.float32)
        for jc in range(_NB):
            s_j = score_r[:, jc * _RC:(jc + 1) * _RC]
            if jc < cb:
                acc = acc + (s_j <= s_i).astype(jnp.float32)
            elif jc > cb:
                acc = acc + (s_j < s_i).astype(jnp.float32)
            else:
                acc = acc + ((s_j < s_i).astype(jnp.float32)
                             + (s_j == s_i).astype(jnp.float32) * tri)
        rank_cols.append(jnp.sum(acc, axis=1, keepdims=True))
    rank_c = jnp.concatenate(rank_cols, axis=0)
    boff = (pl.program_id(0) * W).astype(jnp.float32)
    rank_ref[0, 0, :] = jnp.transpose(rank_c + boff).reshape(W).astype(jnp.int32)


def _encode_u_kernel(u_ref, We_ref, be_ref, W1_ref, b1_ref, W2_ref, b2_ref,
                     att_ref, rec_ref):
    ub = u_ref[0]
    ux = _gelu(lax.dot_general(
        ub, We_ref[...], (((1,), (0,)), ((), ())),
        preferred_element_type=jnp.float32) + be_ref[...])
    gu = lax.dot_general(ub, ub, (((0,), (0,)), ((), ())),
                         preferred_element_type=jnp.float32)
    att_ref[0] = _softmax(gu / float(U))
    h = _gelu(lax.dot_general(
        ux, W1_ref[...], (((1,), (0,)), ((), ())),
        preferred_element_type=jnp.float32) + b1_ref[...])
    z = lax.dot_general(
        h, W2_ref[...], (((1,), (0,)), ((), ())),
        preferred_element_type=jnp.float32) + b2_ref[...]
    rec_ref[0] = 1.0 / (1.0 + jnp.exp(-z))


def _sc_scatter(xflat, grank):
    mesh = plsc.VectorSubcoreMesh(core_axis_name="c", subcore_axis_name="s")
    NCH = W // _GCHUNK

    @functools.partial(
        pl.kernel,
        mesh=mesh,
        out_type=jax.ShapeDtypeStruct((B * W, C), jnp.float32),
        scratch_types=[
            pltpu.VMEM((NCH, _GCHUNK), jnp.int32),
            pltpu.VMEM((2, _GCHUNK, C), jnp.float32),
            pltpu.SemaphoreType.DMA,
            pltpu.SemaphoreType.DMA,
            pltpu.SemaphoreType.DMA,
            pltpu.SemaphoreType.DMA,
        ],
    )
    def scat_k(x_hbm, grank_hbm, out_hbm, ridx_v, rows_v, l0, l1, s0, s1):
        wid = lax.axis_index("s") * 2 + lax.axis_index("c")
        pltpu.sync_copy(grank_hbm.at[pl.ds(wid * NCH, NCH)], ridx_v)
        lsem = (l0, l1)
        ssem = (s0, s1)
        loads = [None, None]
        stores = [None, None]
        loads[0] = pltpu.async_copy(
            x_hbm.at[pl.ds(wid * W, _GCHUNK)], rows_v.at[0], lsem[0])
        for k in range(NCH):
            cur = k % 2
            nxt = (k + 1) % 2
            if k + 1 < NCH:
                if stores[nxt] is not None:
                    stores[nxt].wait()
                loads[nxt] = pltpu.async_copy(
                    x_hbm.at[pl.ds(wid * W + (k + 1) * _GCHUNK, _GCHUNK)],
                    rows_v.at[nxt], lsem[nxt])
            loads[cur].wait()
            stores[cur] = pltpu.async_copy(
                rows_v.at[cur], out_hbm.at[ridx_v.at[k]], ssem[cur])
        stores[0].wait()
        stores[1].wait()

    return scat_k(xflat, grank)


def kernel(x, W1, b1, W2, b2, We, be):
    be2 = be.reshape(1, C)
    rank, out, cg = pl.pallas_call(
        _select_x_kernel,
        grid=(B,),
        in_specs=[
            pl.BlockSpec((1, W, C), lambda b: (b, 0, 0)),
            pl.BlockSpec((C, C), lambda b: (0, 0)),
            pl.BlockSpec((1, C), lambda b: (0, 0)),
        ],
        out_specs=[
            pl.BlockSpec((1, 1, W), lambda b: (b, 0, 0)),
            pl.BlockSpec((1, W, C), lambda b: (b, 0, 0)),
            pl.BlockSpec((1, C, C), lambda b: (b, 0, 0)),
        ],
        out_shape=[
            jax.ShapeDtypeStruct((B, 1, W), jnp.int32),
            jax.ShapeDtypeStruct((B, W, C), jnp.float32),
            jax.ShapeDtypeStruct((B, C, C), jnp.float32),
        ],
        scratch_shapes=[pltpu.VMEM((W + S, C), jnp.float32)],
    )(x, We, be2)

    scat = _sc_scatter(x.reshape(B * W, C),
                       rank.reshape(B * (W // _GCHUNK), _GCHUNK))
    unm = scat.reshape(B, W, C)

    att, rec = pl.pallas_call(
        _encode_u_kernel,
        grid=(B,),
        in_specs=[
            pl.BlockSpec((1, U, C), lambda b: (b, 0, 0)),
            pl.BlockSpec((C, C), lambda b: (0, 0)),
            pl.BlockSpec((1, C), lambda b: (0, 0)),
            pl.BlockSpec((C, C), lambda b: (0, 0)),
            pl.BlockSpec((1, C), lambda b: (0, 0)),
            pl.BlockSpec((C, C), lambda b: (0, 0)),
            pl.BlockSpec((1, C), lambda b: (0, 0)),
        ],
        out_specs=[
            pl.BlockSpec((1, C, C), lambda b: (b, 0, 0)),
            pl.BlockSpec((1, U, C), lambda b: (b, 0, 0)),
        ],
        out_shape=[
            jax.ShapeDtypeStruct((B, C, C), jnp.float32),
            jax.ShapeDtypeStruct((B, U, C), jnp.float32),
        ],
    )(unm, We, be2, W1, b1.reshape(1, C), W2, b2.reshape(1, C))
    return (att, rec, out, cg)


# speedup vs baseline: 1.0023x; 1.0023x over previous
import functools

import jax
import jax.numpy as jnp
from jax import lax
from jax.experimental import pallas as pl
from jax.experimental.pallas import tpu as pltpu
from jax.experimental.pallas import tpu_sc as plsc

B, W, C, S = 32, 2048, 128, 32
TR = W // 2
U = W - TR
_SQRT2 = 1.4142135623730951
_RC = 256
_NB = W // _RC
_GCHUNK = 128
_NGC = U // _GCHUNK


def _gelu(v):
    return v * 0.5 * (1.0 + lax.erf(v / _SQRT2))


def _softmax(v):
    m = jnp.max(v, axis=-1, keepdims=True)
    e = jnp.exp(v - m)
    return e / jnp.sum(e, axis=-1, keepdims=True)


def _win32(a):
    for k in (1, 2, 4, 8, 16):
        z = jnp.zeros((k, a.shape[1]), a.dtype)
        a = a + jnp.concatenate([z, a[: a.shape[0] - k]], axis=0)
    return a


def _win32_scratch(a, scr):
    scr[:S] = jnp.zeros((S, a.shape[1]), a.dtype)
    scr[S:] = a
    for k in (1, 2, 4, 8, 16):
        scr[S:] = scr[S:] + scr[S - k:S - k + W]
    return scr[S:]


def _bf16_rne(v):
    u = lax.bitcast_convert_type(v, jnp.int32)
    r = u + 0x7FFF + jnp.bitwise_and(lax.shift_right_logical(u, 16), 1)
    return lax.bitcast_convert_type(jnp.bitwise_and(r, jnp.int32(-65536)),
                                    jnp.float32)


def _div_den(a):
    wpos = lax.broadcasted_iota(jnp.int32, (S, 1), 0).astype(jnp.float32)
    den = jnp.minimum(wpos + 1.0, float(S))
    return jnp.concatenate([a[:S] / den, a[S:] * (1.0 / S)], axis=0)


def _select_x_kernel(x_ref, We_ref, be_ref, rank_ref, out_ref, cg_ref, scr_ref):
    xb = x_ref[0]

    out_ref[0] = _gelu(lax.dot_general(
        xb, We_ref[...], (((1,), (0,)), ((), ())),
        preferred_element_type=jnp.float32) + be_ref[...])
    gx = lax.dot_general(xb, xb, (((0,), (0,)), ((), ())),
                         preferred_element_type=jnp.float32)
    cg_ref[0] = _softmax(gx / float(W))

    xq = _bf16_rne(xb)
    x2q = _bf16_rne(xb * xb)
    del scr_ref
    ltrm = _div_den(_win32(xq))
    ltr2 = _div_den(_win32(x2q))
    ltrd = ltr2 - ltrm * ltrm
    num = jnp.sum(ltrd, axis=1, keepdims=True)
    dnm = jnp.sum(ltrm, axis=1, keepdims=True)
    score_c = num / dnm
    score_r = jnp.transpose(score_c)

    ii = lax.broadcasted_iota(jnp.int32, (_RC, _RC), 0)
    jj = lax.broadcasted_iota(jnp.int32, (_RC, _RC), 1)
    tri = (jj < ii).astype(jnp.float32)
    rank_cols = []
    for cb in range(_NB):
        s_i = score_c[cb * _RC:(cb + 1) * _RC]
        acc = jnp.zeros((_RC, _RC), jnp.float32)
        for jc in range(_NB):
            s_j = score_r[:, jc * _RC:(jc + 1) * _RC]
            if jc < cb:
                acc = acc + (s_j <= s_i).astype(jnp.float32)
            elif jc > cb:
                acc = acc + (s_j < s_i).astype(jnp.float32)
            else:
                acc = acc + ((s_j < s_i).astype(jnp.float32)
                             + (s_j == s_i).astype(jnp.float32) * tri)
        rank_cols.append(jnp.sum(acc, axis=1, keepdims=True))
    rank_c = jnp.concatenate(rank_cols, axis=0)
    boff = (pl.program_id(0) * W).astype(jnp.float32)
    rank_ref[0, 0, :] = jnp.transpose(rank_c + boff).reshape(W).astype(jnp.int32)


def _encode_u_kernel(u_ref, We_ref, be_ref, W1_ref, b1_ref, W2_ref, b2_ref,
                     att_ref, rec_ref):
    ub = u_ref[0]
    ux = _gelu(lax.dot_general(
        ub, We_ref[...], (((1,), (0,)), ((), ())),
        preferred_element_type=jnp.float32) + be_ref[...])
    gu = lax.dot_general(ub, ub, (((0,), (0,)), ((), ())),
                         preferred_element_type=jnp.float32)
    att_ref[0] = _softmax(gu / float(U))
    h = _gelu(lax.dot_general(
        ux, W1_ref[...], (((1,), (0,)), ((), ())),
        preferred_element_type=jnp.float32) + b1_ref[...])
    z = lax.dot_general(
        h, W2_ref[...], (((1,), (0,)), ((), ())),
        preferred_element_type=jnp.float32) + b2_ref[...]
    rec_ref[0] = 1.0 / (1.0 + jnp.exp(-z))


def _sc_scatter(xflat, grank):
    mesh = plsc.VectorSubcoreMesh(core_axis_name="c", subcore_axis_name="s")
    NCH = W // _GCHUNK

    @functools.partial(
        pl.kernel,
        mesh=mesh,
        out_type=jax.ShapeDtypeStruct((B * W, C), jnp.float32),
        scratch_types=[
            pltpu.VMEM((NCH, _GCHUNK), jnp.int32),
            pltpu.VMEM((2, _GCHUNK, C), jnp.float32),
            pltpu.SemaphoreType.DMA,
            pltpu.SemaphoreType.DMA,
            pltpu.SemaphoreType.DMA,
            pltpu.SemaphoreType.DMA,
        ],
    )
    def scat_k(x_hbm, grank_hbm, out_hbm, ridx_v, rows_v, l0, l1, s0, s1):
        wid = lax.axis_index("s") * 2 + lax.axis_index("c")
        pltpu.sync_copy(grank_hbm.at[pl.ds(wid * NCH, NCH)], ridx_v)
        lsem = (l0, l1)
        ssem = (s0, s1)
        loads = [None, None]
        stores = [None, None]
        loads[0] = pltpu.async_copy(
            x_hbm.at[pl.ds(wid * W, _GCHUNK)], rows_v.at[0], lsem[0])
        for k in range(NCH):
            cur = k % 2
            nxt = (k + 1) % 2
            if k + 1 < NCH:
                if stores[nxt] is not None:
                    stores[nxt].wait()
                loads[nxt] = pltpu.async_copy(
                    x_hbm.at[pl.ds(wid * W + (k + 1) * _GCHUNK, _GCHUNK)],
                    rows_v.at[nxt], lsem[nxt])
            loads[cur].wait()
            stores[cur] = pltpu.async_copy(
                rows_v.at[cur], out_hbm.at[ridx_v.at[k]], ssem[cur])
        stores[0].wait()
        stores[1].wait()

    return scat_k(xflat, grank)


def kernel(x, W1, b1, W2, b2, We, be):
    be2 = be.reshape(1, C)
    rank, out, cg = pl.pallas_call(
        _select_x_kernel,
        grid=(B,),
        in_specs=[
            pl.BlockSpec((1, W, C), lambda b: (b, 0, 0)),
            pl.BlockSpec((C, C), lambda b: (0, 0)),
            pl.BlockSpec((1, C), lambda b: (0, 0)),
        ],
        out_specs=[
            pl.BlockSpec((1, 1, W), lambda b: (b, 0, 0)),
            pl.BlockSpec((1, W, C), lambda b: (b, 0, 0)),
            pl.BlockSpec((1, C, C), lambda b: (b, 0, 0)),
        ],
        out_shape=[
            jax.ShapeDtypeStruct((B, 1, W), jnp.int32),
            jax.ShapeDtypeStruct((B, W, C), jnp.float32),
            jax.ShapeDtypeStruct((B, C, C), jnp.float32),
        ],
        scratch_shapes=[pltpu.VMEM((W + S, C), jnp.float32)],
    )(x, We, be2)

    scat = _sc_scatter(x.reshape(B * W, C),
                       rank.reshape(B * (W // _GCHUNK), _GCHUNK))
    unm = scat.reshape(B, W, C)

    att, rec = pl.pallas_call(
        _encode_u_kernel,
        grid=(B,),
        in_specs=[
            pl.BlockSpec((1, U, C), lambda b: (b, 0, 0)),
            pl.BlockSpec((C, C), lambda b: (0, 0)),
            pl.BlockSpec((1, C), lambda b: (0, 0)),
            pl.BlockSpec((C, C), lambda b: (0, 0)),
            pl.BlockSpec((1, C), lambda b: (0, 0)),
            pl.BlockSpec((C, C), lambda b: (0, 0)),
            pl.BlockSpec((1, C), lambda b: (0, 0)),
        ],
        out_specs=[
            pl.BlockSpec((1, C, C), lambda b: (b, 0, 0)),
            pl.BlockSpec((1, U, C), lambda b: (b, 0, 0)),
        ],
        out_shape=[
            jax.ShapeDtypeStruct((B, C, C), jnp.float32),
            jax.ShapeDtypeStruct((B, U, C), jnp.float32),
        ],
    )(unm, We, be2, W1, b1.reshape(1, C), W2, b2.reshape(1, C))
    return (att, rec, out, cg)


# speedup vs baseline: 1.0885x; 1.0860x over previous
import functools

import jax
import jax.numpy as jnp
from jax import lax
from jax.experimental import pallas as pl
from jax.experimental.pallas import tpu as pltpu
from jax.experimental.pallas import tpu_sc as plsc

B, W, C, S = 32, 2048, 128, 32
TR = W // 2
U = W - TR
_SQRT2 = 1.4142135623730951
_RC = 256
_NB = W // _RC
_GCHUNK = 128
_NGC = U // _GCHUNK


def _gelu(v):
    return v * 0.5 * (1.0 + lax.erf(v / _SQRT2))


def _softmax(v):
    m = jnp.max(v, axis=-1, keepdims=True)
    e = jnp.exp(v - m)
    return e / jnp.sum(e, axis=-1, keepdims=True)


def _win32(a):
    for k in (1, 2, 4, 8, 16):
        z = jnp.zeros((k, a.shape[1]), a.dtype)
        a = a + jnp.concatenate([z, a[: a.shape[0] - k]], axis=0)
    return a


def _win32_scratch(a, scr):
    scr[:S] = jnp.zeros((S, a.shape[1]), a.dtype)
    scr[S:] = a
    for k in (1, 2, 4, 8, 16):
        scr[S:] = scr[S:] + scr[S - k:S - k + W]
    return scr[S:]


def _bf16_rne(v):
    u = lax.bitcast_convert_type(v, jnp.int32)
    r = u + 0x7FFF + jnp.bitwise_and(lax.shift_right_logical(u, 16), 1)
    return lax.bitcast_convert_type(jnp.bitwise_and(r, jnp.int32(-65536)),
                                    jnp.float32)


def _div_den(a):
    wpos = lax.broadcasted_iota(jnp.int32, (S, 1), 0).astype(jnp.float32)
    den = jnp.minimum(wpos + 1.0, float(S))
    return jnp.concatenate([a[:S] / den, a[S:] * (1.0 / S)], axis=0)


def _select_x_kernel(x_ref, We_ref, be_ref, rank_ref, out_ref, cg_ref, scr_ref):
    xb = x_ref[0]

    out_ref[0] = _gelu(lax.dot_general(
        xb, We_ref[...], (((1,), (0,)), ((), ())),
        preferred_element_type=jnp.float32) + be_ref[...])
    gx = lax.dot_general(xb, xb, (((0,), (0,)), ((), ())),
                         preferred_element_type=jnp.float32)
    cg_ref[0] = _softmax(gx / float(W))

    xq = xb.astype(jnp.bfloat16).astype(jnp.float32)
    x2q = (xb * xb).astype(jnp.bfloat16).astype(jnp.float32)
    del scr_ref
    ltrm = _div_den(_win32(xq))
    ltr2 = _div_den(_win32(x2q))
    ltrd = ltr2 - ltrm * ltrm
    num = jnp.sum(ltrd, axis=1, keepdims=True)
    dnm = jnp.sum(ltrm, axis=1, keepdims=True)
    score_c = num / dnm
    score_r = jnp.transpose(score_c)

    ii = lax.broadcasted_iota(jnp.int32, (_RC, _RC), 0)
    jj = lax.broadcasted_iota(jnp.int32, (_RC, _RC), 1)
    tri = (jj < ii).astype(jnp.float32)
    row_cnt = [jnp.zeros((_RC, 1), jnp.float32) for _ in range(_NB)]
    col_cnt = [jnp.zeros((1, _RC), jnp.float32) for _ in range(_NB)]
    for cb in range(_NB):
        s_i = score_c[cb * _RC:(cb + 1) * _RC]
        s_d = score_r[:, cb * _RC:(cb + 1) * _RC]
        d = ((s_d < s_i).astype(jnp.float32)
             + (s_d == s_i).astype(jnp.float32) * tri)
        row_cnt[cb] = row_cnt[cb] + jnp.sum(d, axis=1, keepdims=True)
        for jc in range(cb + 1, _NB):
            s_j = score_r[:, jc * _RC:(jc + 1) * _RC]
            cmp = (s_j < s_i).astype(jnp.float32)
            row_cnt[cb] = row_cnt[cb] + jnp.sum(cmp, axis=1, keepdims=True)
            col_cnt[jc] = col_cnt[jc] + jnp.sum(cmp, axis=0, keepdims=True)
    rank_rows = []
    for c in range(_NB):
        base = float(c * _RC)
        rank_rows.append(jnp.transpose(row_cnt[c]) + (base - col_cnt[c]))
    rank_r = jnp.concatenate(rank_rows, axis=1)
    boff = (pl.program_id(0) * W).astype(jnp.float32)
    rank_ref[0, 0, :] = (rank_r + boff).reshape(W).astype(jnp.int32)


def _encode_u_kernel(u_ref, We_ref, be_ref, W1_ref, b1_ref, W2_ref, b2_ref,
                     att_ref, rec_ref):
    ub = u_ref[0]
    ux = _gelu(lax.dot_general(
        ub, We_ref[...], (((1,), (0,)), ((), ())),
        preferred_element_type=jnp.float32) + be_ref[...])
    gu = lax.dot_general(ub, ub, (((0,), (0,)), ((), ())),
                         preferred_element_type=jnp.float32)
    att_ref[0] = _softmax(gu / float(U))
    h = _gelu(lax.dot_general(
        ux, W1_ref[...], (((1,), (0,)), ((), ())),
        preferred_element_type=jnp.float32) + b1_ref[...])
    z = lax.dot_general(
        h, W2_ref[...], (((1,), (0,)), ((), ())),
        preferred_element_type=jnp.float32) + b2_ref[...]
    rec_ref[0] = 1.0 / (1.0 + jnp.exp(-z))


def _sc_scatter(xflat, grank):
    mesh = plsc.VectorSubcoreMesh(core_axis_name="c", subcore_axis_name="s")
    NCH = W // _GCHUNK

    @functools.partial(
        pl.kernel,
        mesh=mesh,
        out_type=jax.ShapeDtypeStruct((B * W, C), jnp.float32),
        scratch_types=[
            pltpu.VMEM((NCH, _GCHUNK), jnp.int32),
            pltpu.VMEM((2, _GCHUNK, C), jnp.float32),
            pltpu.SemaphoreType.DMA,
            pltpu.SemaphoreType.DMA,
            pltpu.SemaphoreType.DMA,
            pltpu.SemaphoreType.DMA,
        ],
    )
    def scat_k(x_hbm, grank_hbm, out_hbm, ridx_v, rows_v, l0, l1, s0, s1):
        wid = lax.axis_index("s") * 2 + lax.axis_index("c")
        pltpu.sync_copy(grank_hbm.at[pl.ds(wid * NCH, NCH)], ridx_v)
        lsem = (l0, l1)
        ssem = (s0, s1)
        loads = [None, None]
        stores = [None, None]
        loads[0] = pltpu.async_copy(
            x_hbm.at[pl.ds(wid * W, _GCHUNK)], rows_v.at[0], lsem[0])
        for k in range(NCH):
            cur = k % 2
            nxt = (k + 1) % 2
            if k + 1 < NCH:
                if stores[nxt] is not None:
                    stores[nxt].wait()
                loads[nxt] = pltpu.async_copy(
                    x_hbm.at[pl.ds(wid * W + (k + 1) * _GCHUNK, _GCHUNK)],
                    rows_v.at[nxt], lsem[nxt])
            loads[cur].wait()
            stores[cur] = pltpu.async_copy(
                rows_v.at[cur], out_hbm.at[ridx_v.at[k]], ssem[cur])
        stores[0].wait()
        stores[1].wait()

    return scat_k(xflat, grank)


def kernel(x, W1, b1, W2, b2, We, be):
    be2 = be.reshape(1, C)
    rank, out, cg = pl.pallas_call(
        _select_x_kernel,
        grid=(B,),
        in_specs=[
            pl.BlockSpec((1, W, C), lambda b: (b, 0, 0)),
            pl.BlockSpec((C, C), lambda b: (0, 0)),
            pl.BlockSpec((1, C), lambda b: (0, 0)),
        ],
        out_specs=[
            pl.BlockSpec((1, 1, W), lambda b: (b, 0, 0)),
            pl.BlockSpec((1, W, C), lambda b: (b, 0, 0)),
            pl.BlockSpec((1, C, C), lambda b: (b, 0, 0)),
        ],
        out_shape=[
            jax.ShapeDtypeStruct((B, 1, W), jnp.int32),
            jax.ShapeDtypeStruct((B, W, C), jnp.float32),
            jax.ShapeDtypeStruct((B, C, C), jnp.float32),
        ],
        scratch_shapes=[pltpu.VMEM((W + S, C), jnp.float32)],
    )(x, We, be2)

    scat = _sc_scatter(x.reshape(B * W, C),
                       rank.reshape(B * (W // _GCHUNK), _GCHUNK))
    unm = scat.reshape(B, W, C)

    att, rec = pl.pallas_call(
        _encode_u_kernel,
        grid=(B,),
        in_specs=[
            pl.BlockSpec((1, U, C), lambda b: (b, 0, 0)),
            pl.BlockSpec((C, C), lambda b: (0, 0)),
            pl.BlockSpec((1, C), lambda b: (0, 0)),
            pl.BlockSpec((C, C), lambda b: (0, 0)),
            pl.BlockSpec((1, C), lambda b: (0, 0)),
            pl.BlockSpec((C, C), lambda b: (0, 0)),
            pl.BlockSpec((1, C), lambda b: (0, 0)),
        ],
        out_specs=[
            pl.BlockSpec((1, C, C), lambda b: (b, 0, 0)),
            pl.BlockSpec((1, U, C), lambda b: (b, 0, 0)),
        ],
        out_shape=[
            jax.ShapeDtypeStruct((B, C, C), jnp.float32),
            jax.ShapeDtypeStruct((B, U, C), jnp.float32),
        ],
    )(unm, We, be2, W1, b1.reshape(1, C), W2, b2.reshape(1, C))
    return (att, rec, out, cg)
